# Initial kernel scaffold; baseline (speedup 1.0000x reference)
import functools

import jax
import jax.numpy as jnp
from jax import lax
from jax.experimental import pallas as pl
from jax.experimental.pallas import tpu as pltpu
from jax.experimental.pallas import tpu_sc as plsc

_H, _W = 20, 20
_D = _H * _W
_VOCAB = 8
_ROWS = 64 * 32 * 16
_NW = 32
_RPW = _ROWS // _NW
_CH = 128
_NCH = _RPW // _CH
_L = 16


def _body(x_hbm, embed_hbm, out_hbm, x_v, idx_v, buf_v, sem):
    cid = lax.axis_index("c")
    sid = lax.axis_index("s")
    wid = sid * 2 + cid
    base = wid * _RPW

    pltpu.sync_copy(x_hbm.at[pl.ds(base, _RPW)], x_v)

    lanes = lax.iota(jnp.int32, _L)

    def compute_idx_group(c, g):
        rows = jnp.full((_L,), c * _CH + g * _L, jnp.int32) + lanes
        best = plsc.load_gather(x_v, [rows, jnp.zeros((_L,), jnp.int32)])
        bi = jnp.zeros((_L,), jnp.int32)
        for j in range(1, _VOCAB):
            v = plsc.load_gather(x_v, [rows, jnp.full((_L,), j, jnp.int32)])
            m = v > best
            best = jnp.where(m, v, best)
            bi = jnp.where(m, jnp.full((_L,), j, jnp.int32), bi)
        idx_v[c, pl.ds(g * _L, _L)] = bi

    for c in range(_NCH):
        for g in range(_CH // _L):
            compute_idx_group(c, g)
        pltpu.async_copy(embed_hbm.at[idx_v.at[c]], buf_v, sem).wait()
        pltpu.sync_copy(buf_v, out_hbm.at[pl.ds(base + c * _CH, _CH)])


@jax.jit
def _encode(x_flat, embed):
    mesh = plsc.VectorSubcoreMesh(core_axis_name="c", subcore_axis_name="s")
    return pl.kernel(
        _body,
        out_type=jax.ShapeDtypeStruct((_ROWS, _D), jnp.float32),
        mesh=mesh,
        scratch_types=[
            pltpu.VMEM((_RPW, _VOCAB), jnp.float32),
            pltpu.VMEM((_NCH, _CH), jnp.int32),
            pltpu.VMEM((_CH, _D), jnp.float32),
            pltpu.SemaphoreType.DMA,
        ],
    )(x_flat, embed)


def kernel(x, embed):
    b, n, c, _ = x.shape
    out = _encode(x.reshape(_ROWS, _VOCAB), embed)
    return out.reshape(b * n, c, _H, _W)


# speedup vs baseline: 1.1343x; 1.1343x over previous
import functools

import jax
import jax.numpy as jnp
from jax import lax
from jax.experimental import pallas as pl
from jax.experimental.pallas import tpu as pltpu
from jax.experimental.pallas import tpu_sc as plsc

_H, _W = 20, 20
_D = _H * _W
_VOCAB = 8
_ROWS = 64 * 32 * 16
_NW = 32
_RPW = _ROWS // _NW
_CH = 128
_NCH = _RPW // _CH
_L = 16


def _body(x_hbm, embed_hbm, out_hbm, x_v, idx_v, buf_v, sem):
    cid = lax.axis_index("c")
    sid = lax.axis_index("s")
    wid = sid * 2 + cid
    base = wid * _RPW

    pltpu.sync_copy(x_hbm.at[pl.ds(base * _VOCAB, _RPW * _VOCAB)], x_v)

    lanes = lax.iota(jnp.int32, _L)

    def compute_idx_group(c, g):
        rows = jnp.full((_L,), c * _CH + g * _L, jnp.int32) + lanes
        fbase = rows * _VOCAB
        best = plsc.load_gather(x_v, [fbase])
        bi = jnp.zeros((_L,), jnp.int32)
        for j in range(1, _VOCAB):
            v = plsc.load_gather(x_v, [fbase + j])
            m = v > best
            best = jnp.where(m, v, best)
            bi = jnp.where(m, jnp.full((_L,), j, jnp.int32), bi)
        idx_v[c, pl.ds(g * _L, _L)] = bi

    for c in range(_NCH):
        for g in range(_CH // _L):
            compute_idx_group(c, g)
        pltpu.async_copy(embed_hbm.at[idx_v.at[c]], buf_v, sem).wait()
        pltpu.sync_copy(buf_v, out_hbm.at[pl.ds(base + c * _CH, _CH)])


@jax.jit
def _encode(x_flat, embed):
    mesh = plsc.VectorSubcoreMesh(core_axis_name="c", subcore_axis_name="s")
    return pl.kernel(
        _body,
        out_type=jax.ShapeDtypeStruct((_ROWS, _D), jnp.float32),
        mesh=mesh,
        compiler_params=pltpu.CompilerParams(
            needs_layout_passes=False, use_tc_tiling_on_sc=False),
        scratch_types=[
            pltpu.VMEM((_RPW * _VOCAB,), jnp.float32),
            pltpu.VMEM((_NCH, _CH), jnp.int32),
            pltpu.VMEM((_CH, _D), jnp.float32),
            pltpu.SemaphoreType.DMA,
        ],
    )(x_flat, embed)


def kernel(x, embed):
    b, n, c, _ = x.shape
    out = _encode(x.reshape(_ROWS * _VOCAB), embed)
    return out.reshape(b * n, c, _H, _W)


# speedup vs baseline: 2.6932x; 2.3743x over previous
import functools

import jax
import jax.numpy as jnp
from jax import lax
from jax.experimental import pallas as pl
from jax.experimental.pallas import tpu as pltpu
from jax.experimental.pallas import tpu_sc as plsc

_H, _W = 20, 20
_VOCAB = 8
_B, _N, _C = 64, 32, 16
_ROWS = _B * _N * _C
_OUTROWS = _B * _N
_NW = 32
_L = 16

_BPW = _B // _NW
_NHALF = 16

_ORPW = _OUTROWS // _NW

_mesh = plsc.VectorSubcoreMesh(core_axis_name="c", subcore_axis_name="s")
_cparams = pltpu.CompilerParams(needs_layout_passes=False,
                                use_tc_tiling_on_sc=True)


def _argmax_body(x_hbm, idx_hbm, x_v, idx_v):
    wid = lax.axis_index("s") * 2 + lax.axis_index("c")
    b0 = wid * _BPW
    lanes = lax.iota(jnp.int32, _L)
    zeros = jnp.zeros((_L,), jnp.int32)

    for chunk in range(_BPW * 2):
        b = b0 + chunk // 2
        n0 = (chunk % 2) * _NHALF
        pltpu.sync_copy(x_hbm.at[b, pl.ds(n0, _NHALF)], x_v)
        for n in range(_NHALF):
            best = plsc.load_gather(x_v, [jnp.full((_L,), n, jnp.int32),
                                          lanes, zeros])
            bi = zeros
            for j in range(1, _VOCAB):
                v = plsc.load_gather(x_v, [jnp.full((_L,), n, jnp.int32),
                                           lanes,
                                           jnp.full((_L,), j, jnp.int32)])
                m = v > best
                best = jnp.where(m, v, best)
                bi = jnp.where(m, jnp.full((_L,), j, jnp.int32), bi)
            idx_v[pl.ds((chunk * _NHALF + n) * _C, _C)] = bi
    pltpu.sync_copy(idx_v, idx_hbm.at[pl.ds(wid * (_BPW * _N * _C),
                                            _BPW * _N * _C)])


def _lookup_body(table_hbm, idx_hbm, out_hbm, tc_v, idx_v, osem):
    wid = lax.axis_index("s") * 2 + lax.axis_index("c")
    r0 = wid * _ORPW

    pltpu.sync_copy(table_hbm, tc_v)
    pltpu.sync_copy(idx_hbm.at[pl.ds(r0 * _C, _ORPW * _C)], idx_v)

    def plane_idx(p):
        v = plsc.load_gather(idx_v, [jnp.full((_L,), p, jnp.int32)])
        return jnp.max(v)

    def fire_row(k):
        for c in range(_C):
            i = plane_idx(k * _C + c)
            pltpu.async_copy(tc_v.at[i], out_hbm.at[r0 + k, c], osem)

    def drain_row(k):
        for c in range(_C):
            pltpu.make_async_copy(tc_v.at[0], out_hbm.at[r0 + k, c],
                                  osem).wait()

    lag = 4
    def body(k):
        fire_row(k)
        @pl.when(k >= lag)
        def _():
            drain_row(k - lag)

    pl.loop(0, _ORPW)(body)
    pl.loop(_ORPW - lag, _ORPW)(drain_row)


@jax.jit
def _encode(x, table3):
    idx = pl.kernel(
        _argmax_body,
        out_type=jax.ShapeDtypeStruct((_ROWS,), jnp.int32),
        mesh=_mesh,
        compiler_params=_cparams,
        scratch_types=[
            pltpu.VMEM((_NHALF, _C, _VOCAB), jnp.float32),
            pltpu.VMEM((_BPW * _N * _C,), jnp.int32),
        ],
    )(x)
    out = pl.kernel(
        _lookup_body,
        out_type=jax.ShapeDtypeStruct((_OUTROWS, _C, _H, _W), jnp.float32),
        mesh=_mesh,
        compiler_params=_cparams,
        scratch_types=[
            pltpu.VMEM((_VOCAB, _H, _W), jnp.float32),
            pltpu.VMEM((_ORPW * _C,), jnp.int32),
            pltpu.SemaphoreType.DMA,
        ],
    )(table3, idx)
    return out


def kernel(x, embed):
    return _encode(x, embed.reshape(_VOCAB, _H, _W))


# speedup vs baseline: 3.8614x; 1.4337x over previous
import functools

import jax
import jax.numpy as jnp
from jax import lax
from jax.experimental import pallas as pl
from jax.experimental.pallas import tpu as pltpu
from jax.experimental.pallas import tpu_sc as plsc

_H, _W = 20, 20
_VOCAB = 8
_B, _N, _C = 64, 32, 16
_ROWS = _B * _N * _C
_OUTROWS = _B * _N
_NW = 32
_L = 16

_BPW = _B // _NW
_NHALF = 16

_ORPW = _OUTROWS // _NW

_mesh = plsc.VectorSubcoreMesh(core_axis_name="c", subcore_axis_name="s")
_cparams = pltpu.CompilerParams(needs_layout_passes=False,
                                use_tc_tiling_on_sc=True)


def _argmax_body(x_hbm, idx_hbm, x_v, idx_v):
    wid = lax.axis_index("s") * 2 + lax.axis_index("c")
    b0 = wid * _BPW
    lanes = lax.iota(jnp.int32, _L)
    zeros = jnp.zeros((_L,), jnp.int32)

    for chunk in range(_BPW * 2):
        b = b0 + chunk // 2
        n0 = (chunk % 2) * _NHALF
        pltpu.sync_copy(x_hbm.at[b, pl.ds(n0, _NHALF)], x_v)
        for n in range(_NHALF):
            best = plsc.load_gather(x_v, [jnp.full((_L,), n, jnp.int32),
                                          lanes, zeros])
            bi = zeros
            for j in range(1, _VOCAB):
                v = plsc.load_gather(x_v, [jnp.full((_L,), n, jnp.int32),
                                           lanes,
                                           jnp.full((_L,), j, jnp.int32)])
                m = v > best
                best = jnp.where(m, v, best)
                bi = jnp.where(m, jnp.full((_L,), j, jnp.int32), bi)
            idx_v[pl.ds((chunk * _NHALF + n) * _C, _C)] = bi
    pltpu.sync_copy(idx_v, idx_hbm.at[pl.ds(wid * (_BPW * _N * _C),
                                            _BPW * _N * _C)])


_NRB = 16
_RB = _OUTROWS // _NRB
_SLABS = _H * _W
_SPH = _SLABS // 2
_VPS = _C * _RB // _L


def _lookup_body(table_hbm, idx_hbm, out_hbm, table_v, idxs_v, idxoff_v,
                 buf0, buf1, osem0, osem1):
    wid = lax.axis_index("s") * 2 + lax.axis_index("c")
    rblk = wid % _NRB
    s0 = (wid // _NRB) * _SPH
    lanes = lax.iota(jnp.int32, _L)

    pltpu.sync_copy(table_hbm, table_v)
    pltpu.sync_copy(idx_hbm.at[pl.ds(rblk * _RB * _C, _RB * _C)], idxs_v)

    for ct in range(2):
        for cc in range(8):
            c = ct * 8 + cc
            for rv in range(_RB // _L):
                g = plsc.load_gather(
                    idxs_v, [(jnp.full((_L,), rv * _L, jnp.int32) + lanes)
                             * _C + c])
                vpos = (ct * 8 + cc) * (_RB // _L) + rv
                idxoff_v[pl.ds(vpos * _L, _L)] = g * (_H * _W)

    bufs = (buf0, buf1)
    osems = (osem0, osem1)

    def out_at(s):
        return out_hbm.at[s // _W, s % _W, slice(None),
                          pl.ds(rblk * _RB, _RB)]

    def do_slab(k, buf, osem, drain):
        s = s0 + k
        if drain:
            pltpu.make_async_copy(buf, out_at(s - 2), osem).wait()
        sv = jnp.full((_L,), s, jnp.int32)
        for ct in range(2):
            for cc in range(8):
                for rv in range(_RB // _L):
                    vpos = (ct * 8 + cc) * (_RB // _L) + rv
                    val = plsc.load_gather(
                        table_v, [idxoff_v[pl.ds(vpos * _L, _L)] + sv])
                    buf[ct * 8 + cc, pl.ds(rv * _L, _L)] = val
        pltpu.async_copy(buf, out_at(s), osem)

    do_slab(0, buf0, osem0, False)
    do_slab(1, buf1, osem1, False)

    def pair(k):
        do_slab(k, buf0, osem0, True)
        do_slab(k + 1, buf1, osem1, True)

    pl.loop(2, _SPH, step=2)(pair)
    pltpu.make_async_copy(buf0, out_at(s0 + _SPH - 2), osem0).wait()
    pltpu.make_async_copy(buf1, out_at(s0 + _SPH - 1), osem1).wait()


@jax.jit
def _encode(x, table3):
    idx = pl.kernel(
        _argmax_body,
        out_type=jax.ShapeDtypeStruct((_ROWS,), jnp.int32),
        mesh=_mesh,
        compiler_params=_cparams,
        scratch_types=[
            pltpu.VMEM((_NHALF, _C, _VOCAB), jnp.float32),
            pltpu.VMEM((_BPW * _N * _C,), jnp.int32),
        ],
    )(x)
    out_t = pl.kernel(
        _lookup_body,
        out_type=jax.ShapeDtypeStruct((_H, _W, _C, _OUTROWS), jnp.float32),
        mesh=_mesh,
        compiler_params=_cparams,
        scratch_types=[
            pltpu.VMEM((_VOCAB * _H * _W,), jnp.float32),
            pltpu.VMEM((_RB * _C,), jnp.int32),
            pltpu.VMEM((_RB * _C,), jnp.int32),
            pltpu.VMEM((_C, _RB), jnp.float32),
            pltpu.VMEM((_C, _RB), jnp.float32),
            pltpu.SemaphoreType.DMA,
            pltpu.SemaphoreType.DMA,
        ],
    )(table3, idx)
    return out_t


def kernel(x, embed):
    out_t = _encode(x, embed.reshape(_VOCAB * _H * _W))
    return jnp.transpose(out_t, (3, 2, 0, 1))


# speedup vs baseline: 6.2181x; 1.6103x over previous
import functools

import jax
import jax.numpy as jnp
from jax import lax
from jax.experimental import pallas as pl
from jax.experimental.pallas import tpu as pltpu
from jax.experimental.pallas import tpu_sc as plsc

_H, _W = 20, 20
_VOCAB = 8
_B, _N, _C = 64, 32, 16
_ROWS = _B * _N * _C
_OUTROWS = _B * _N
_NW = 32
_L = 16

_BPW = _B // _NW
_NHALF = 16

_ORPW = _OUTROWS // _NW

_mesh = plsc.VectorSubcoreMesh(core_axis_name="c", subcore_axis_name="s")
_cparams = pltpu.CompilerParams(needs_layout_passes=False,
                                use_tc_tiling_on_sc=True)


def _argmax_body(x_hbm, idx_hbm, x_v, idx_v):
    wid = lax.axis_index("s") * 2 + lax.axis_index("c")
    b0 = wid * _BPW
    lanes = lax.iota(jnp.int32, _L)
    zeros = jnp.zeros((_L,), jnp.int32)

    for chunk in range(_BPW * 2):
        b = b0 + chunk // 2
        n0 = (chunk % 2) * _NHALF
        pltpu.sync_copy(x_hbm.at[b, pl.ds(n0, _NHALF)], x_v)
        for n in range(_NHALF):
            best = plsc.load_gather(x_v, [jnp.full((_L,), n, jnp.int32),
                                          lanes, zeros])
            bi = zeros
            for j in range(1, _VOCAB):
                v = plsc.load_gather(x_v, [jnp.full((_L,), n, jnp.int32),
                                           lanes,
                                           jnp.full((_L,), j, jnp.int32)])
                m = v > best
                best = jnp.where(m, v, best)
                bi = jnp.where(m, jnp.full((_L,), j, jnp.int32), bi)
            idx_v[pl.ds((chunk * _NHALF + n) * _C, _C)] = bi
    pltpu.sync_copy(idx_v, idx_hbm.at[pl.ds(wid * (_BPW * _N * _C),
                                            _BPW * _N * _C)])


_NRB = 16
_RB = _OUTROWS // _NRB
_SLABS = _H * _W
_SPH = _SLABS // 2
_VPS = _C * _RB // _L
_K = 4


def _lookup_body(table_hbm, idx_hbm, out_hbm, table_v, idxs_v, idxoff_v,
                 buf0, buf1, osem0, osem1):
    wid = lax.axis_index("s") * 2 + lax.axis_index("c")
    rblk = wid % _NRB
    s0 = (wid // _NRB) * _SPH
    lanes = lax.iota(jnp.int32, _L)

    pltpu.sync_copy(table_hbm, table_v)
    pltpu.sync_copy(idx_hbm.at[pl.ds(rblk * _RB * _C, _RB * _C)], idxs_v)

    for ct in range(2):
        for cc in range(8):
            c = ct * 8 + cc
            for rv in range(_RB // _L):
                g = plsc.load_gather(
                    idxs_v, [(jnp.full((_L,), rv * _L, jnp.int32) + lanes)
                             * _C + c])
                vpos = (ct * 8 + cc) * (_RB // _L) + rv
                idxoff_v[pl.ds(vpos * _L, _L)] = g * (_H * _W)

    bufs = (buf0, buf1)
    osems = (osem0, osem1)

    def out_at(s):
        return out_hbm.at[s // _W, s % _W, slice(None),
                          pl.ds(rblk * _RB, _RB)]

    def do_group(k, buf, osem, drain):
        s = s0 + k
        if drain:
            for j in range(_K):
                pltpu.make_async_copy(buf.at[j], out_at(s - 2 * _K + j),
                                      osem).wait()
        svs = [jnp.full((_L,), s, jnp.int32) + j for j in range(_K)]
        for vp in range(0, _VPS, 2):
            ix = [idxoff_v[pl.ds((vp + g) * _L, _L)] for g in range(2)]
            vals = [plsc.load_gather(table_v, [ix[g] + svs[j]])
                    for g in range(2) for j in range(_K)]
            for g in range(2):
                c, rv = (vp + g) // (_RB // _L), (vp + g) % (_RB // _L)
                for j in range(_K):
                    buf[j, c, pl.ds(rv * _L, _L)] = vals[g * _K + j]
        for j in range(_K):
            pltpu.async_copy(buf.at[j], out_at(s + j), osem)

    do_group(0, buf0, osem0, False)
    do_group(_K, buf1, osem1, False)

    def pair(k):
        do_group(k, buf0, osem0, True)
        do_group(k + _K, buf1, osem1, True)

    pl.loop(2 * _K, _SPH, step=2 * _K)(pair)
    for j in range(_K):
        pltpu.make_async_copy(buf0.at[j], out_at(s0 + _SPH - 2 * _K + j),
                              osem0).wait()
        pltpu.make_async_copy(buf1.at[j], out_at(s0 + _SPH - _K + j),
                              osem1).wait()


@jax.jit
def _encode(x, table3):
    idx = pl.kernel(
        _argmax_body,
        out_type=jax.ShapeDtypeStruct((_ROWS,), jnp.int32),
        mesh=_mesh,
        compiler_params=_cparams,
        scratch_types=[
            pltpu.VMEM((_NHALF, _C, _VOCAB), jnp.float32),
            pltpu.VMEM((_BPW * _N * _C,), jnp.int32),
        ],
    )(x)
    out_t = pl.kernel(
        _lookup_body,
        out_type=jax.ShapeDtypeStruct((_H, _W, _C, _OUTROWS), jnp.float32),
        mesh=_mesh,
        compiler_params=_cparams,
        scratch_types=[
            pltpu.VMEM((_VOCAB * _H * _W,), jnp.float32),
            pltpu.VMEM((_RB * _C,), jnp.int32),
            pltpu.VMEM((_RB * _C,), jnp.int32),
            pltpu.VMEM((_K, _C, _RB), jnp.float32),
            pltpu.VMEM((_K, _C, _RB), jnp.float32),
            pltpu.SemaphoreType.DMA,
            pltpu.SemaphoreType.DMA,
        ],
    )(table3, idx)
    return out_t


def kernel(x, embed):
    out_t = _encode(x, embed.reshape(_VOCAB * _H * _W))
    return jnp.transpose(out_t, (3, 2, 0, 1))


# speedup vs baseline: 6.2536x; 1.0057x over previous
import functools

import jax
import jax.numpy as jnp
from jax import lax
from jax.experimental import pallas as pl
from jax.experimental.pallas import tpu as pltpu
from jax.experimental.pallas import tpu_sc as plsc

_H, _W = 20, 20
_VOCAB = 8
_B, _N, _C = 64, 32, 16
_ROWS = _B * _N * _C
_OUTROWS = _B * _N
_NW = 32
_L = 16

_BPW = _B // _NW
_NHALF = 16

_ORPW = _OUTROWS // _NW

_mesh = plsc.VectorSubcoreMesh(core_axis_name="c", subcore_axis_name="s")
_cparams = pltpu.CompilerParams(needs_layout_passes=False,
                                use_tc_tiling_on_sc=True)


def _argmax_body(x_hbm, idx_hbm, x_v, idx_v):
    wid = lax.axis_index("s") * 2 + lax.axis_index("c")
    b0 = wid * _BPW
    lanes = lax.iota(jnp.int32, _L)
    zeros = jnp.zeros((_L,), jnp.int32)

    for chunk in range(_BPW * 2):
        b = b0 + chunk // 2
        n0 = (chunk % 2) * _NHALF
        pltpu.sync_copy(x_hbm.at[b, pl.ds(n0, _NHALF)], x_v)
        for n in range(_NHALF):
            best = plsc.load_gather(x_v, [jnp.full((_L,), n, jnp.int32),
                                          lanes, zeros])
            bi = zeros
            for j in range(1, _VOCAB):
                v = plsc.load_gather(x_v, [jnp.full((_L,), n, jnp.int32),
                                           lanes,
                                           jnp.full((_L,), j, jnp.int32)])
                m = v > best
                best = jnp.where(m, v, best)
                bi = jnp.where(m, jnp.full((_L,), j, jnp.int32), bi)
            idx_v[pl.ds((chunk * _NHALF + n) * _C, _C)] = bi
    pltpu.sync_copy(idx_v, idx_hbm.at[pl.ds(wid * (_BPW * _N * _C),
                                            _BPW * _N * _C)])


_NRB = 16
_RB = _OUTROWS // _NRB
_SLABS = _H * _W
_SPH = _SLABS // 2
_VPS = _C * _RB // _L
_K = 4


def _lookup_body(table_hbm, idx_hbm, out_hbm, table_v, idxs_v, idxoff_v,
                 buf0, buf1, osem0, osem1):
    wid = lax.axis_index("s") * 2 + lax.axis_index("c")
    rblk = wid % _NRB
    s0 = (wid // _NRB) * _SPH
    lanes = lax.iota(jnp.int32, _L)

    pltpu.sync_copy(table_hbm, table_v)
    pltpu.sync_copy(idx_hbm.at[pl.ds(rblk * _RB * _C, _RB * _C)], idxs_v)

    for ct in range(2):
        for cc in range(8):
            c = ct * 8 + cc
            for rv in range(_RB // _L):
                g = plsc.load_gather(
                    idxs_v, [(jnp.full((_L,), rv * _L, jnp.int32) + lanes)
                             * _C + c])
                vpos = (ct * 8 + cc) * (_RB // _L) + rv
                idxoff_v[pl.ds(vpos * _L, _L)] = g * (_H * _W)

    bufs = (buf0, buf1)
    osems = (osem0, osem1)

    def out_at(s):
        return out_hbm.at[s // _W, pl.ds(s % _W, _K), slice(None),
                          pl.ds(rblk * _RB, _RB)]

    def do_group(k, buf, osem, drain):
        s = s0 + k
        if drain:
            pltpu.make_async_copy(buf, out_at(s - 2 * _K), osem).wait()
        svs = [jnp.full((_L,), s, jnp.int32) + j for j in range(_K)]
        for vp in range(0, _VPS, 2):
            ix = [idxoff_v[pl.ds((vp + g) * _L, _L)] for g in range(2)]
            vals = [plsc.load_gather(table_v, [ix[g] + svs[j]])
                    for g in range(2) for j in range(_K)]
            for g in range(2):
                c, rv = (vp + g) // (_RB // _L), (vp + g) % (_RB // _L)
                for j in range(_K):
                    buf[j, c, pl.ds(rv * _L, _L)] = vals[g * _K + j]
        pltpu.async_copy(buf, out_at(s), osem)

    do_group(0, buf0, osem0, False)
    do_group(_K, buf1, osem1, False)

    def pair(k):
        do_group(k, buf0, osem0, True)
        do_group(k + _K, buf1, osem1, True)

    pl.loop(2 * _K, _SPH, step=2 * _K)(pair)
    pltpu.make_async_copy(buf0, out_at(s0 + _SPH - 2 * _K), osem0).wait()
    pltpu.make_async_copy(buf1, out_at(s0 + _SPH - _K), osem1).wait()


@jax.jit
def _encode(x, table3):
    idx = pl.kernel(
        _argmax_body,
        out_type=jax.ShapeDtypeStruct((_ROWS,), jnp.int32),
        mesh=_mesh,
        compiler_params=_cparams,
        scratch_types=[
            pltpu.VMEM((_NHALF, _C, _VOCAB), jnp.float32),
            pltpu.VMEM((_BPW * _N * _C,), jnp.int32),
        ],
    )(x)
    out_t = pl.kernel(
        _lookup_body,
        out_type=jax.ShapeDtypeStruct((_H, _W, _C, _OUTROWS), jnp.float32),
        mesh=_mesh,
        compiler_params=_cparams,
        scratch_types=[
            pltpu.VMEM((_VOCAB * _H * _W,), jnp.float32),
            pltpu.VMEM((_RB * _C,), jnp.int32),
            pltpu.VMEM((_RB * _C,), jnp.int32),
            pltpu.VMEM((_K, _C, _RB), jnp.float32),
            pltpu.VMEM((_K, _C, _RB), jnp.float32),
            pltpu.SemaphoreType.DMA,
            pltpu.SemaphoreType.DMA,
        ],
    )(table3, idx)
    return out_t


def kernel(x, embed):
    out_t = _encode(x, embed.reshape(_VOCAB * _H * _W))
    return jnp.transpose(out_t, (3, 2, 0, 1))
